# baseline (speedup 1.0000x reference)
import jax
import jax.numpy as jnp
from jax import lax
from jax.experimental import pallas as pl
from jax.experimental.pallas import tpu as pltpu
from jax.experimental.pallas import tpu_sc as plsc

_NUM_CORES = 2
_NUM_SUBCORES = 16
_NUM_WORKERS = _NUM_CORES * _NUM_SUBCORES
_LANE_PAD = 128
_LANES = 16


def kernel(session_week_id, emb_weight):
    batch = session_week_id.shape[0]
    dim = emb_weight.shape[1]
    b_per_w = batch // _NUM_WORKERS
    idx = session_week_id.astype(jnp.int32)
    table = jnp.pad(emb_weight, ((0, 0), (0, _LANE_PAD - dim)))

    n_chunks = 4
    chunk = b_per_w // n_chunks

    mesh = plsc.VectorSubcoreMesh(core_axis_name="c", subcore_axis_name="s")

    @pl.kernel(
        out_type=jax.ShapeDtypeStruct((batch, dim), emb_weight.dtype),
        mesh=mesh,
        scratch_types=[
            pltpu.VMEM((b_per_w,), jnp.int32),
            pltpu.VMEM((chunk, _LANE_PAD), emb_weight.dtype),
            pltpu.VMEM((chunk, _LANE_PAD), emb_weight.dtype),
            pltpu.VMEM((chunk, dim), emb_weight.dtype),
            pltpu.VMEM((chunk, dim), emb_weight.dtype),
            pltpu.SemaphoreType.DMA,
            pltpu.SemaphoreType.DMA,
            pltpu.SemaphoreType.DMA,
            pltpu.SemaphoreType.DMA,
        ],
    )
    def _gather(
        table_hbm, idx_hbm, out_hbm, idx_v, a0, a1, b0, b1, g0, g1, w0, w1
    ):
        wid = lax.axis_index("s") * _NUM_CORES + lax.axis_index("c")
        base = wid * b_per_w
        pltpu.sync_copy(idx_hbm.at[pl.ds(base, b_per_w)], idx_v)
        bufs_a = (a0, a1)
        bufs_b = (b0, b1)
        gsems = (g0, g1)
        wsems = (w0, w1)
        gather_handles = [None, None]
        write_handles = [None, None]
        gather_handles[0] = pltpu.async_copy(
            table_hbm.at[idx_v.at[pl.ds(0, chunk)]], bufs_a[0], gsems[0]
        )
        for k in range(n_chunks):
            b = k % 2
            gather_handles[b].wait()
            if k + 1 < n_chunks:
                gather_handles[1 - b] = pltpu.async_copy(
                    table_hbm.at[idx_v.at[pl.ds((k + 1) * chunk, chunk)]],
                    bufs_a[1 - b],
                    gsems[1 - b],
                )
            if write_handles[b] is not None:
                write_handles[b].wait()
            a_ref = bufs_a[b]
            b_ref = bufs_b[b]

            @pl.loop(0, chunk)
            def _(j):
                for s in range(dim // _LANES):
                    b_ref[j, pl.ds(s * _LANES, _LANES)] = a_ref[
                        j, pl.ds(s * _LANES, _LANES)
                    ]

            write_handles[b] = pltpu.async_copy(
                b_ref, out_hbm.at[pl.ds(base + k * chunk, chunk)], wsems[b]
            )
        write_handles[0].wait()
        write_handles[1].wait()

    return _gather(table, idx)
